# baseline (speedup 1.0000x reference)
import functools

import jax
import jax.numpy as jnp
from jax import lax
from jax.experimental import pallas as pl
from jax.experimental.pallas import tpu as pltpu
from jax.experimental.pallas import tpu_sc as plsc

_ROWS = 6400
_NW = 32
_ROWS_PER_W = 8


def _make_sc_copy():
    mesh = plsc.VectorSubcoreMesh(core_axis_name="c", subcore_axis_name="s")

    @functools.partial(
        pl.kernel,
        mesh=mesh,
        out_type=jax.ShapeDtypeStruct((_ROWS, 128), jnp.int32),
        scratch_types=[
            pltpu.VMEM((_ROWS_PER_W, 128), jnp.int32),
        ],
    )
    def sc_copy(x_hbm, out_hbm, buf):
        wid = lax.axis_index("s") * 2 + lax.axis_index("c")
        rows = pl.ds(wid * _ROWS_PER_W, _ROWS_PER_W)
        pltpu.sync_copy(x_hbm.at[rows], buf)
        pltpu.sync_copy(buf, out_hbm.at[rows])

    return sc_copy


_sc_copy = _make_sc_copy()


def kernel(z, x, W_h, b_h, emb):
    del z, W_h, b_h, emb
    x2 = jnp.reshape(x, (_ROWS, 128))
    out = _sc_copy(x2)
    return jnp.reshape(out, (4096, 200))


# speedup vs baseline: 2.5275x; 2.5275x over previous
import jax
import jax.numpy as jnp
from jax.experimental import pallas as pl
from jax.experimental.pallas import tpu as pltpu

_BATCH = 4096
_HIST = 200


def _body(x_hbm, o_hbm, buf, sem_in, sem_out):
    cin = pltpu.make_async_copy(x_hbm, buf, sem_in)
    cin.start()
    cin.wait()
    cout = pltpu.make_async_copy(buf, o_hbm, sem_out)
    cout.start()
    cout.wait()


def kernel(z, x, W_h, b_h, emb):
    del z, W_h, b_h, emb
    return pl.pallas_call(
        _body,
        out_shape=jax.ShapeDtypeStruct((_BATCH, _HIST), jnp.int32),
        in_specs=[pl.BlockSpec(memory_space=pl.MemorySpace.ANY)],
        out_specs=pl.BlockSpec(memory_space=pl.MemorySpace.ANY),
        scratch_shapes=[
            pltpu.VMEM((_BATCH, _HIST), jnp.int32),
            pltpu.SemaphoreType.DMA,
            pltpu.SemaphoreType.DMA,
        ],
    )(x)


# speedup vs baseline: 2.5596x; 1.0127x over previous
import jax
import jax.numpy as jnp
from jax.experimental import pallas as pl
from jax.experimental.pallas import tpu as pltpu

_BATCH = 4096
_HIST = 200
_ROW_BLOCK = 2048


def _copy_body(x_ref, o_ref):
    o_ref[...] = x_ref[...]


def kernel(z, x, W_h, b_h, emb):
    del z, W_h, b_h, emb
    grid = (_BATCH // _ROW_BLOCK,)
    return pl.pallas_call(
        _copy_body,
        out_shape=jax.ShapeDtypeStruct((_BATCH, _HIST), jnp.int32),
        grid=grid,
        in_specs=[pl.BlockSpec((_ROW_BLOCK, _HIST), lambda i: (i, 0))],
        out_specs=pl.BlockSpec((_ROW_BLOCK, _HIST), lambda i: (i, 0)),
        compiler_params=pltpu.CompilerParams(
            dimension_semantics=("arbitrary",),
        ),
    )(x)
